# baseline (device time: 193586 ns/iter reference)
import jax
import jax.numpy as jnp
from jax import lax
from jax.experimental import pallas as pl
from jax.experimental.pallas import tpu as pltpu

N_DEV = 8


def kernel(x, w_mat, scale_x, scale_w):
    m_global, k_per = x.shape
    _, n = w_mat.shape
    m_per = m_global // N_DEV

    xb = x.astype(jnp.bfloat16)
    wb = w_mat.astype(jnp.bfloat16)

    def body(x_ref, w_ref, sx_ref, sw_ref, out_ref,
             send_buf, recv_bufs, send_sem, recv_sems):
        me = lax.axis_index("i")
        left = lax.rem(me + N_DEV - 1, N_DEV)
        right = lax.rem(me + 1, N_DEV)

        barrier_sem = pltpu.get_barrier_semaphore()
        for nbr in (left, right):
            pl.semaphore_signal(
                barrier_sem, inc=1,
                device_id=(nbr,), device_id_type=pl.DeviceIdType.MESH,
            )
        pl.semaphore_wait(barrier_sem, 2)

        def partial(c):
            xblk = x_ref[pl.ds(c * m_per, m_per), :]
            return jnp.dot(xblk, w_ref[:, :],
                           preferred_element_type=jnp.float32)

        acc = None
        for s in range(N_DEV - 1):
            c = lax.rem(me + (2 * N_DEV - 1 - s), N_DEV)
            acc = partial(c)
            if s > 0:
                acc = acc + recv_bufs[s - 1, :, :].astype(jnp.float32)
            send_buf[:, :] = acc.astype(jnp.bfloat16)
            rdma = pltpu.make_async_remote_copy(
                src_ref=send_buf,
                dst_ref=recv_bufs.at[s],
                send_sem=send_sem,
                recv_sem=recv_sems.at[s],
                device_id=(right,),
                device_id_type=pl.DeviceIdType.MESH,
            )
            rdma.start()
            rdma.wait()

        final = partial(me) + recv_bufs[N_DEV - 2, :, :].astype(jnp.float32)
        y = final * (sx_ref[0] * sw_ref[0])
        yc = jnp.clip(y, -60.0, 60.0)
        out_ref[:, :] = y * (1.0 / (1.0 + jnp.exp(-yc)))

    return pl.pallas_call(
        body,
        out_shape=jax.ShapeDtypeStruct((m_per, n), jnp.float32),
        in_specs=[
            pl.BlockSpec(memory_space=pltpu.VMEM),
            pl.BlockSpec(memory_space=pltpu.VMEM),
            pl.BlockSpec(memory_space=pltpu.SMEM),
            pl.BlockSpec(memory_space=pltpu.SMEM),
        ],
        out_specs=pl.BlockSpec(memory_space=pltpu.VMEM),
        scratch_shapes=[
            pltpu.VMEM((m_per, n), jnp.bfloat16),
            pltpu.VMEM((N_DEV - 1, m_per, n), jnp.bfloat16),
            pltpu.SemaphoreType.DMA,
            pltpu.SemaphoreType.DMA((N_DEV - 1,)),
        ],
        compiler_params=pltpu.CompilerParams(collective_id=0),
    )(xb, wb, scale_x, scale_w)


# device time: 118160 ns/iter; 1.6383x vs baseline; 1.6383x over previous
import jax
import jax.numpy as jnp
from jax import lax
from jax.experimental import pallas as pl
from jax.experimental.pallas import tpu as pltpu

N_DEV = 8


def kernel(x, w_mat, scale_x, scale_w):
    m_global, k_per = x.shape
    _, n = w_mat.shape
    m_per = m_global // N_DEV
    nh = n // 2

    xb = x.astype(jnp.bfloat16)
    wb = w_mat.astype(jnp.bfloat16)

    def body(x_ref, w_ref, sx_ref, sw_ref, out_ref,
             send_cw, send_ccw, recv_cw, recv_ccw,
             send_sems, recv_sems_cw, recv_sems_ccw):
        me = lax.axis_index("i")
        left = lax.rem(me + N_DEV - 1, N_DEV)
        right = lax.rem(me + 1, N_DEV)

        barrier_sem = pltpu.get_barrier_semaphore()
        for nbr in (left, right):
            pl.semaphore_signal(
                barrier_sem, inc=1,
                device_id=(nbr,), device_id_type=pl.DeviceIdType.MESH,
            )
        pl.semaphore_wait(barrier_sem, 2)

        def partial_cw(c):
            xblk = x_ref[pl.ds(c * m_per, m_per), :]
            return jnp.dot(xblk, w_ref[:, :nh],
                           preferred_element_type=jnp.float32)

        def partial_ccw(c):
            xblk = x_ref[pl.ds(c * m_per, m_per), :]
            return jnp.dot(xblk, w_ref[:, nh:],
                           preferred_element_type=jnp.float32)

        def start_step(s):
            r1 = pltpu.make_async_remote_copy(
                src_ref=send_cw, dst_ref=recv_cw.at[s],
                send_sem=send_sems.at[0], recv_sem=recv_sems_cw.at[s],
                device_id=(right,), device_id_type=pl.DeviceIdType.MESH,
            )
            r2 = pltpu.make_async_remote_copy(
                src_ref=send_ccw, dst_ref=recv_ccw.at[s],
                send_sem=send_sems.at[1], recv_sem=recv_sems_ccw.at[s],
                device_id=(left,), device_id_type=pl.DeviceIdType.MESH,
            )
            r1.start()
            r2.start()
            return r1, r2

        def c_cw(s):
            return lax.rem(me + (2 * N_DEV - 1 - s), N_DEV)

        def c_ccw(s):
            return lax.rem(me + 1 + s, N_DEV)

        send_cw[:, :] = partial_cw(c_cw(0)).astype(jnp.bfloat16)
        send_ccw[:, :] = partial_ccw(c_ccw(0)).astype(jnp.bfloat16)
        rdmas = start_step(0)

        for s in range(N_DEV - 1):
            if s < N_DEV - 2:
                p_cw = partial_cw(c_cw(s + 1))
                p_ccw = partial_ccw(c_ccw(s + 1))
            else:
                p_cw = partial_cw(me)
                p_ccw = partial_ccw(me)
            rdmas[0].wait()
            rdmas[1].wait()
            if s < N_DEV - 2:
                send_cw[:, :] = (
                    p_cw + recv_cw[s, :, :].astype(jnp.float32)
                ).astype(jnp.bfloat16)
                send_ccw[:, :] = (
                    p_ccw + recv_ccw[s, :, :].astype(jnp.float32)
                ).astype(jnp.bfloat16)
                rdmas = start_step(s + 1)

        fin_cw = p_cw + recv_cw[N_DEV - 2, :, :].astype(jnp.float32)
        fin_ccw = p_ccw + recv_ccw[N_DEV - 2, :, :].astype(jnp.float32)
        sc = sx_ref[0] * sw_ref[0]
        y1 = fin_cw * sc
        y2 = fin_ccw * sc
        out_ref[:, :nh] = y1 * (1.0 / (1.0 + jnp.exp(-jnp.clip(y1, -60.0, 60.0))))
        out_ref[:, nh:] = y2 * (1.0 / (1.0 + jnp.exp(-jnp.clip(y2, -60.0, 60.0))))

    return pl.pallas_call(
        body,
        out_shape=jax.ShapeDtypeStruct((m_per, n), jnp.float32),
        in_specs=[
            pl.BlockSpec(memory_space=pltpu.VMEM),
            pl.BlockSpec(memory_space=pltpu.VMEM),
            pl.BlockSpec(memory_space=pltpu.SMEM),
            pl.BlockSpec(memory_space=pltpu.SMEM),
        ],
        out_specs=pl.BlockSpec(memory_space=pltpu.VMEM),
        scratch_shapes=[
            pltpu.VMEM((m_per, nh), jnp.bfloat16),
            pltpu.VMEM((m_per, nh), jnp.bfloat16),
            pltpu.VMEM((N_DEV - 1, m_per, nh), jnp.bfloat16),
            pltpu.VMEM((N_DEV - 1, m_per, nh), jnp.bfloat16),
            pltpu.SemaphoreType.DMA((2,)),
            pltpu.SemaphoreType.DMA((N_DEV - 1,)),
            pltpu.SemaphoreType.DMA((N_DEV - 1,)),
        ],
        compiler_params=pltpu.CompilerParams(collective_id=0),
    )(xb, wb, scale_x, scale_w)


# device time: 94187 ns/iter; 2.0553x vs baseline; 1.2545x over previous
import jax
import jax.numpy as jnp
from jax import lax
from jax.experimental import pallas as pl
from jax.experimental.pallas import tpu as pltpu

N_DEV = 8
N_TILE = 2


def kernel(x, w_mat, scale_x, scale_w):
    m_global, k_per = x.shape
    _, n = w_mat.shape
    m_per = m_global // N_DEV
    nh = n // 2
    tm = m_per // N_TILE

    xb = x.astype(jnp.bfloat16)
    wb = w_mat.astype(jnp.bfloat16)

    def body(x_ref, w_ref, sx_ref, sw_ref, out_ref,
             send_cw, send_ccw, recv_cw, recv_ccw,
             send_sems_cw, send_sems_ccw, recv_sems_cw, recv_sems_ccw):
        me = lax.axis_index("i")
        left = lax.rem(me + N_DEV - 1, N_DEV)
        right = lax.rem(me + 1, N_DEV)

        barrier_sem = pltpu.get_barrier_semaphore()
        for nbr in (left, right):
            pl.semaphore_signal(
                barrier_sem, inc=1,
                device_id=(nbr,), device_id_type=pl.DeviceIdType.MESH,
            )
        pl.semaphore_wait(barrier_sem, 2)

        def partial_cw(c):
            xblk = x_ref[pl.ds(c * m_per, m_per), :]
            return jnp.dot(xblk, w_ref[:, :nh],
                           preferred_element_type=jnp.float32)

        def partial_ccw(c):
            xblk = x_ref[pl.ds(c * m_per, m_per), :]
            return jnp.dot(xblk, w_ref[:, nh:],
                           preferred_element_type=jnp.float32)

        def c_cw(s):
            return lax.rem(me + (2 * N_DEV - 1 - s), N_DEV)

        def c_ccw(s):
            return lax.rem(me + 1 + s, N_DEV)

        def make_rdma(dir_cw, s, t):
            if dir_cw:
                return pltpu.make_async_remote_copy(
                    src_ref=send_cw.at[t], dst_ref=recv_cw.at[s, t],
                    send_sem=send_sems_cw.at[t],
                    recv_sem=recv_sems_cw.at[s, t],
                    device_id=(right,), device_id_type=pl.DeviceIdType.MESH,
                )
            return pltpu.make_async_remote_copy(
                src_ref=send_ccw.at[t], dst_ref=recv_ccw.at[s, t],
                send_sem=send_sems_ccw.at[t],
                recv_sem=recv_sems_ccw.at[s, t],
                device_id=(left,), device_id_type=pl.DeviceIdType.MESH,
            )

        p_cw = partial_cw(c_cw(0))
        p_ccw = partial_ccw(c_ccw(0))
        rdmas = {}
        for t in range(N_TILE):
            send_cw[t] = p_cw[t * tm:(t + 1) * tm, :].astype(jnp.bfloat16)
            send_ccw[t] = p_ccw[t * tm:(t + 1) * tm, :].astype(jnp.bfloat16)
            rdmas[(True, t)] = make_rdma(True, 0, t)
            rdmas[(False, t)] = make_rdma(False, 0, t)
            rdmas[(True, t)].start()
            rdmas[(False, t)].start()

        for s in range(1, N_DEV - 1):
            p_cw = partial_cw(c_cw(s))
            p_ccw = partial_ccw(c_ccw(s))
            for t in range(N_TILE):
                for dc, p, send, recv in (
                    (True, p_cw, send_cw, recv_cw),
                    (False, p_ccw, send_ccw, recv_ccw),
                ):
                    prev = rdmas[(dc, t)]
                    prev.wait_recv()
                    acc = (p[t * tm:(t + 1) * tm, :]
                           + recv[s - 1, t].astype(jnp.float32))
                    prev.wait_send()
                    send[t] = acc.astype(jnp.bfloat16)
                    nxt = make_rdma(dc, s, t)
                    nxt.start()
                    rdmas[(dc, t)] = nxt

        p_cw = partial_cw(me)
        p_ccw = partial_ccw(me)
        sc = sx_ref[0] * sw_ref[0]
        for t in range(N_TILE):
            for dc, p, recv, col0 in (
                (True, p_cw, recv_cw, 0),
                (False, p_ccw, recv_ccw, nh),
            ):
                rdmas[(dc, t)].wait_recv()
                fin = (p[t * tm:(t + 1) * tm, :]
                       + recv[N_DEV - 2, t].astype(jnp.float32))
                y = fin * sc
                out_ref[pl.ds(t * tm, tm), col0:col0 + nh] = (
                    y * (1.0 / (1.0 + jnp.exp(-jnp.clip(y, -60.0, 60.0))))
                )
        for t in range(N_TILE):
            rdmas[(True, t)].wait_send()
            rdmas[(False, t)].wait_send()

    return pl.pallas_call(
        body,
        out_shape=jax.ShapeDtypeStruct((m_per, n), jnp.float32),
        in_specs=[
            pl.BlockSpec(memory_space=pltpu.VMEM),
            pl.BlockSpec(memory_space=pltpu.VMEM),
            pl.BlockSpec(memory_space=pltpu.SMEM),
            pl.BlockSpec(memory_space=pltpu.SMEM),
        ],
        out_specs=pl.BlockSpec(memory_space=pltpu.VMEM),
        scratch_shapes=[
            pltpu.VMEM((N_TILE, tm, nh), jnp.bfloat16),
            pltpu.VMEM((N_TILE, tm, nh), jnp.bfloat16),
            pltpu.VMEM((N_DEV - 1, N_TILE, tm, nh), jnp.bfloat16),
            pltpu.VMEM((N_DEV - 1, N_TILE, tm, nh), jnp.bfloat16),
            pltpu.SemaphoreType.DMA((N_TILE,)),
            pltpu.SemaphoreType.DMA((N_TILE,)),
            pltpu.SemaphoreType.DMA((N_DEV - 1, N_TILE)),
            pltpu.SemaphoreType.DMA((N_DEV - 1, N_TILE)),
        ],
        compiler_params=pltpu.CompilerParams(collective_id=0),
    )(xb, wb, scale_x, scale_w)


# device time: 93891 ns/iter; 2.0618x vs baseline; 1.0032x over previous
import jax
import jax.numpy as jnp
from jax import lax
from jax.experimental import pallas as pl
from jax.experimental.pallas import tpu as pltpu

N_DEV = 8
N_TILE = 2


def kernel(x, w_mat, scale_x, scale_w):
    m_global, k_per = x.shape
    _, n = w_mat.shape
    m_per = m_global // N_DEV
    nh = n // 2
    tm = m_per // N_TILE

    def body(x_ref, w_ref, sx_ref, sw_ref, out_ref,
             wb_ref, send_cw, send_ccw, recv_cw, recv_ccw,
             send_sems_cw, send_sems_ccw, recv_sems_cw, recv_sems_ccw):
        me = lax.axis_index("i")
        left = lax.rem(me + N_DEV - 1, N_DEV)
        right = lax.rem(me + 1, N_DEV)

        barrier_sem = pltpu.get_barrier_semaphore()
        for nbr in (left, right):
            pl.semaphore_signal(
                barrier_sem, inc=1,
                device_id=(nbr,), device_id_type=pl.DeviceIdType.MESH,
            )
        pl.semaphore_wait(barrier_sem, 2)

        wb_ref[:, :] = w_ref[:, :].astype(jnp.bfloat16)

        def partial_cw(c, rows=m_per, roff=0):
            xblk = x_ref[pl.ds(c * m_per + roff, rows), :].astype(jnp.bfloat16)
            return jnp.dot(xblk, wb_ref[:, :nh],
                           preferred_element_type=jnp.float32)

        def partial_ccw(c, rows=m_per, roff=0):
            xblk = x_ref[pl.ds(c * m_per + roff, rows), :].astype(jnp.bfloat16)
            return jnp.dot(xblk, wb_ref[:, nh:],
                           preferred_element_type=jnp.float32)

        def c_cw(s):
            return lax.rem(me + (2 * N_DEV - 1 - s), N_DEV)

        def c_ccw(s):
            return lax.rem(me + 1 + s, N_DEV)

        def make_rdma(dir_cw, s, t):
            if dir_cw:
                return pltpu.make_async_remote_copy(
                    src_ref=send_cw.at[t], dst_ref=recv_cw.at[s, t],
                    send_sem=send_sems_cw.at[t],
                    recv_sem=recv_sems_cw.at[s, t],
                    device_id=(right,), device_id_type=pl.DeviceIdType.MESH,
                )
            return pltpu.make_async_remote_copy(
                src_ref=send_ccw.at[t], dst_ref=recv_ccw.at[s, t],
                send_sem=send_sems_ccw.at[t],
                recv_sem=recv_sems_ccw.at[s, t],
                device_id=(left,), device_id_type=pl.DeviceIdType.MESH,
            )

        rdmas = {}
        for t in range(N_TILE):
            send_cw[t] = partial_cw(
                c_cw(0), rows=tm, roff=t * tm).astype(jnp.bfloat16)
            rdmas[(True, t)] = make_rdma(True, 0, t)
            rdmas[(True, t)].start()
            send_ccw[t] = partial_ccw(
                c_ccw(0), rows=tm, roff=t * tm).astype(jnp.bfloat16)
            rdmas[(False, t)] = make_rdma(False, 0, t)
            rdmas[(False, t)].start()

        for s in range(1, N_DEV - 1):
            p_cw = partial_cw(c_cw(s))
            p_ccw = partial_ccw(c_ccw(s))
            for t in range(N_TILE):
                for dc, p, send, recv in (
                    (True, p_cw, send_cw, recv_cw),
                    (False, p_ccw, send_ccw, recv_ccw),
                ):
                    prev = rdmas[(dc, t)]
                    prev.wait_recv()
                    acc = (p[t * tm:(t + 1) * tm, :]
                           + recv[s - 1, t].astype(jnp.float32))
                    prev.wait_send()
                    send[t] = acc.astype(jnp.bfloat16)
                    nxt = make_rdma(dc, s, t)
                    nxt.start()
                    rdmas[(dc, t)] = nxt

        p_cw = partial_cw(me)
        p_ccw = partial_ccw(me)
        sc = sx_ref[0] * sw_ref[0]
        for t in range(N_TILE):
            for dc, p, recv, col0 in (
                (True, p_cw, recv_cw, 0),
                (False, p_ccw, recv_ccw, nh),
            ):
                rdmas[(dc, t)].wait_recv()
                fin = (p[t * tm:(t + 1) * tm, :]
                       + recv[N_DEV - 2, t].astype(jnp.float32))
                y = fin * sc
                out_ref[pl.ds(t * tm, tm), col0:col0 + nh] = (
                    y * (1.0 / (1.0 + jnp.exp(-jnp.clip(y, -60.0, 60.0))))
                )
        for t in range(N_TILE):
            rdmas[(True, t)].wait_send()
            rdmas[(False, t)].wait_send()

    return pl.pallas_call(
        body,
        out_shape=jax.ShapeDtypeStruct((m_per, n), jnp.float32),
        in_specs=[
            pl.BlockSpec(memory_space=pltpu.VMEM),
            pl.BlockSpec(memory_space=pltpu.VMEM),
            pl.BlockSpec(memory_space=pltpu.SMEM),
            pl.BlockSpec(memory_space=pltpu.SMEM),
        ],
        out_specs=pl.BlockSpec(memory_space=pltpu.VMEM),
        scratch_shapes=[
            pltpu.VMEM((k_per, n), jnp.bfloat16),
            pltpu.VMEM((N_TILE, tm, nh), jnp.bfloat16),
            pltpu.VMEM((N_TILE, tm, nh), jnp.bfloat16),
            pltpu.VMEM((N_DEV - 1, N_TILE, tm, nh), jnp.bfloat16),
            pltpu.VMEM((N_DEV - 1, N_TILE, tm, nh), jnp.bfloat16),
            pltpu.SemaphoreType.DMA((N_TILE,)),
            pltpu.SemaphoreType.DMA((N_TILE,)),
            pltpu.SemaphoreType.DMA((N_DEV - 1, N_TILE)),
            pltpu.SemaphoreType.DMA((N_DEV - 1, N_TILE)),
        ],
        compiler_params=pltpu.CompilerParams(collective_id=0),
    )(x, w_mat, scale_x, scale_w)


# device time: 93689 ns/iter; 2.0663x vs baseline; 1.0022x over previous
import jax
import jax.numpy as jnp
from jax import lax
from jax.experimental import pallas as pl
from jax.experimental.pallas import tpu as pltpu

N_DEV = 8
N_TILE = 4


def kernel(x, w_mat, scale_x, scale_w):
    m_global, k_per = x.shape
    _, n = w_mat.shape
    m_per = m_global // N_DEV
    nh = n // 2
    tm = m_per // N_TILE

    def body(x_ref, w_ref, sx_ref, sw_ref, out_ref,
             wb_ref, send_cw, send_ccw, recv_cw, recv_ccw,
             send_sems_cw, send_sems_ccw, recv_sems_cw, recv_sems_ccw):
        me = lax.axis_index("i")
        left = lax.rem(me + N_DEV - 1, N_DEV)
        right = lax.rem(me + 1, N_DEV)

        barrier_sem = pltpu.get_barrier_semaphore()
        for nbr in (left, right):
            pl.semaphore_signal(
                barrier_sem, inc=1,
                device_id=(nbr,), device_id_type=pl.DeviceIdType.MESH,
            )
        pl.semaphore_wait(barrier_sem, 2)

        wb_ref[:, :] = w_ref[:, :].astype(jnp.bfloat16)

        def partial_cw(c, rows=m_per, roff=0):
            xblk = x_ref[pl.ds(c * m_per + roff, rows), :].astype(jnp.bfloat16)
            return jnp.dot(xblk, wb_ref[:, :nh],
                           preferred_element_type=jnp.float32
                           ).astype(jnp.bfloat16)

        def partial_ccw(c, rows=m_per, roff=0):
            xblk = x_ref[pl.ds(c * m_per + roff, rows), :].astype(jnp.bfloat16)
            return jnp.dot(xblk, wb_ref[:, nh:],
                           preferred_element_type=jnp.float32
                           ).astype(jnp.bfloat16)

        def c_cw(s):
            return lax.rem(me + (2 * N_DEV - 1 - s), N_DEV)

        def c_ccw(s):
            return lax.rem(me + 1 + s, N_DEV)

        def make_rdma(dir_cw, s, t):
            if dir_cw:
                return pltpu.make_async_remote_copy(
                    src_ref=send_cw.at[t], dst_ref=recv_cw.at[s, t],
                    send_sem=send_sems_cw.at[t],
                    recv_sem=recv_sems_cw.at[s, t],
                    device_id=(right,), device_id_type=pl.DeviceIdType.MESH,
                )
            return pltpu.make_async_remote_copy(
                src_ref=send_ccw.at[t], dst_ref=recv_ccw.at[s, t],
                send_sem=send_sems_ccw.at[t],
                recv_sem=recv_sems_ccw.at[s, t],
                device_id=(left,), device_id_type=pl.DeviceIdType.MESH,
            )

        rdmas = {}
        for t in range(N_TILE):
            send_cw[t] = partial_cw(c_cw(0), rows=tm, roff=t * tm)
            rdmas[(True, t)] = make_rdma(True, 0, t)
            rdmas[(True, t)].start()
            send_ccw[t] = partial_ccw(c_ccw(0), rows=tm, roff=t * tm)
            rdmas[(False, t)] = make_rdma(False, 0, t)
            rdmas[(False, t)].start()

        for s in range(1, N_DEV - 1):
            p_cw = partial_cw(c_cw(s))
            p_ccw = partial_ccw(c_ccw(s))
            for t in range(N_TILE):
                for dc, p, send, recv in (
                    (True, p_cw, send_cw, recv_cw),
                    (False, p_ccw, send_ccw, recv_ccw),
                ):
                    prev = rdmas[(dc, t)]
                    prev.wait_recv()
                    acc = p[t * tm:(t + 1) * tm, :] + recv[s - 1, t]
                    prev.wait_send()
                    send[t] = acc
                    nxt = make_rdma(dc, s, t)
                    nxt.start()
                    rdmas[(dc, t)] = nxt

        p_cw = partial_cw(me)
        p_ccw = partial_ccw(me)
        sc = sx_ref[0] * sw_ref[0]
        for t in range(N_TILE):
            for dc, p, recv, col0 in (
                (True, p_cw, recv_cw, 0),
                (False, p_ccw, recv_ccw, nh),
            ):
                rdmas[(dc, t)].wait_recv()
                fin = (p[t * tm:(t + 1) * tm, :].astype(jnp.float32)
                       + recv[N_DEV - 2, t].astype(jnp.float32))
                y = fin * sc
                out_ref[pl.ds(t * tm, tm), col0:col0 + nh] = (
                    y * (1.0 / (1.0 + jnp.exp(-jnp.clip(y, -60.0, 60.0))))
                )
        for t in range(N_TILE):
            rdmas[(True, t)].wait_send()
            rdmas[(False, t)].wait_send()

    return pl.pallas_call(
        body,
        out_shape=jax.ShapeDtypeStruct((m_per, n), jnp.float32),
        in_specs=[
            pl.BlockSpec(memory_space=pltpu.VMEM),
            pl.BlockSpec(memory_space=pltpu.VMEM),
            pl.BlockSpec(memory_space=pltpu.SMEM),
            pl.BlockSpec(memory_space=pltpu.SMEM),
        ],
        out_specs=pl.BlockSpec(memory_space=pltpu.VMEM),
        scratch_shapes=[
            pltpu.VMEM((k_per, n), jnp.bfloat16),
            pltpu.VMEM((N_TILE, tm, nh), jnp.bfloat16),
            pltpu.VMEM((N_TILE, tm, nh), jnp.bfloat16),
            pltpu.VMEM((N_DEV - 1, N_TILE, tm, nh), jnp.bfloat16),
            pltpu.VMEM((N_DEV - 1, N_TILE, tm, nh), jnp.bfloat16),
            pltpu.SemaphoreType.DMA((N_TILE,)),
            pltpu.SemaphoreType.DMA((N_TILE,)),
            pltpu.SemaphoreType.DMA((N_DEV - 1, N_TILE)),
            pltpu.SemaphoreType.DMA((N_DEV - 1, N_TILE)),
        ],
        compiler_params=pltpu.CompilerParams(collective_id=0),
    )(x, w_mat, scale_x, scale_w)
